# baseline (speedup 1.0000x reference)
import functools

import jax
import jax.numpy as jnp
from jax import lax
from jax.experimental import pallas as pl
from jax.experimental.pallas import tpu as pltpu
from jax.experimental.pallas import tpu_sc as plsc

NC = 2
NS = 16
L = 16
NW = NC * NS

N = 10000
E = 320000
D = 128
PER_W = E // NW
CHUNK = 80
N_ITERS = PER_W // CHUNK
RS = 2
KS = 4
UNROLL = 4
N_MAIN = 124
OUT_W = 25 * CHUNK
ZS_STEP = 624


def _sc_body(z_hbm, src_hbm, dst_hbm, out_hbm,
             rows, is_v, id_v, out_v, zsh, sem_g, sem_ix):
    cid = lax.axis_index("c")
    sid = lax.axis_index("s")
    lane = lax.iota(jnp.int32, L)
    base_w = (sid * NC + cid) * PER_W

    zrow = sid * ZS_STEP
    pltpu.sync_copy(z_hbm.at[pl.ds(zrow, ZS_STEP)],
                    zsh.at[pl.ds(zrow, ZS_STEP)])

    @pl.when(sid == NS - 1)
    def _stage_tail():
        pltpu.sync_copy(z_hbm.at[pl.ds(NS * ZS_STEP, N - NS * ZS_STEP)],
                        zsh.at[pl.ds(NS * ZS_STEP, N - NS * ZS_STEP)])

    plsc.subcore_barrier()

    def issue_idx(k, chunk):
        off = base_w + chunk * CHUNK
        pltpu.async_copy(src_hbm.at[pl.ds(off, CHUNK)], is_v[k], sem_ix[k])
        pltpu.async_copy(dst_hbm.at[pl.ds(off, CHUNK)], id_v[k], sem_ix[k])

    def wait_idx(k):
        pltpu.make_async_copy(src_hbm.at[pl.ds(0, CHUNK)], is_v[k],
                              sem_ix[k]).wait()
        pltpu.make_async_copy(dst_hbm.at[pl.ds(0, CHUNK)], id_v[k],
                              sem_ix[k]).wait()

    def issue_g(b, k):
        pltpu.async_copy(zsh.at[is_v[k]], rows[b].at[pl.ds(0, CHUNK)],
                         sem_g[b])
        pltpu.async_copy(zsh.at[id_v[k]], rows[b].at[pl.ds(CHUNK, CHUNK)],
                         sem_g[b])

    def wait_g(b):
        pltpu.make_async_copy(zsh.at[is_v[0]], rows[b].at[pl.ds(0, CHUNK)],
                              sem_g[b]).wait()
        pltpu.make_async_copy(zsh.at[id_v[0]], rows[b].at[pl.ds(CHUNK, CHUNK)],
                              sem_g[b]).wait()

    def compute(b, chunk):
        cbase = (chunk % 25) * CHUNK

        @plsc.parallel_loop(0, CHUNK // L)
        def group_body(g):
            eids = g * L + lane
            tids = eids + CHUNK

            def d_blk(j, accs):
                a0, a1, a2, a3 = accs
                ps = []
                for u in range(16):
                    dv = (lane + (j * 16 + u)) & (D - 1)
                    s = plsc.load_gather(rows[b], [eids, dv])
                    t = plsc.load_gather(rows[b], [tids, dv])
                    ps.append(s * t)
                a0 = a0 + ((ps[0] + ps[1]) + (ps[2] + ps[3]))
                a1 = a1 + ((ps[4] + ps[5]) + (ps[6] + ps[7]))
                a2 = a2 + ((ps[8] + ps[9]) + (ps[10] + ps[11]))
                a3 = a3 + ((ps[12] + ps[13]) + (ps[14] + ps[15]))
                return a0, a1, a2, a3

            z4 = jnp.zeros((L,), jnp.float32)
            a0, a1, a2, a3 = lax.fori_loop(0, D // 16, d_blk,
                                           (z4, z4, z4, z4))
            acc = (a0 + a1) + (a2 + a3)
            out_v[pl.ds(cbase + g * L, L)] = 1.0 / (1.0 + jnp.exp(-acc))

    def step_main(chunk, j):
        b, k = j % RS, j % KS
        wait_g(b)
        compute(b, chunk)

        @pl.when(chunk + 4 < N_ITERS)
        def _ix():
            issue_idx(k, chunk + 4)

        @pl.when(chunk + 2 < N_ITERS)
        def _g():
            wait_idx((j + 2) % KS)
            issue_g(b, (j + 2) % KS)

        @pl.when(chunk % 25 == 24)
        def _flush():
            pltpu.sync_copy(
                out_v,
                out_hbm.at[pl.ds(base_w + (chunk // 25) * OUT_W, OUT_W)])

    for k in range(KS):
        issue_idx(k, k)
    for b in range(RS):
        wait_idx(b)
        issue_g(b, b)

    def outer(o, carry):
        for j in range(UNROLL):
            step_main(o * UNROLL + j, j)
        return carry

    lax.fori_loop(0, N_MAIN // UNROLL, outer, 0)
    wait_g(0)
    compute(0, N_ITERS - 1)
    pltpu.sync_copy(out_v, out_hbm.at[pl.ds(base_w + 4 * OUT_W, OUT_W)])


@jax.jit
def _run(z, src, dst):
    mesh = plsc.VectorSubcoreMesh(
        core_axis_name="c", subcore_axis_name="s",
        num_cores=NC, num_subcores=NS)
    kfn = pl.kernel(
        _sc_body,
        out_type=jax.ShapeDtypeStruct((E,), jnp.float32),
        mesh=mesh,
        scratch_types=[
            [pltpu.VMEM((2 * CHUNK, D), jnp.float32) for _ in range(RS)],
            [pltpu.VMEM((CHUNK,), jnp.int32) for _ in range(KS)],
            [pltpu.VMEM((CHUNK,), jnp.int32) for _ in range(KS)],
            pltpu.VMEM((OUT_W,), jnp.float32),
            pltpu.VMEM_SHARED((N, D), jnp.float32),
            [pltpu.SemaphoreType.DMA for _ in range(RS)],
            [pltpu.SemaphoreType.DMA for _ in range(KS)],
        ],
        compiler_params=pltpu.CompilerParams(needs_layout_passes=False),
    )
    return kfn(z, src, dst)


def kernel(z, edge_index):
    src = edge_index[0].astype(jnp.int32)
    dst = edge_index[1].astype(jnp.int32)
    return _run(z, src, dst)


# speedup vs baseline: 1.1450x; 1.1450x over previous
import functools

import jax
import jax.numpy as jnp
from jax import lax
from jax.experimental import pallas as pl
from jax.experimental.pallas import tpu as pltpu
from jax.experimental.pallas import tpu_sc as plsc

NC = 2
NS = 16
L = 16
NW = NC * NS

E = 320000
D = 128
PER_W = E // NW
CHUNK = 80
N_ITERS = PER_W // CHUNK
N_SLOTS = 5
OUT_W = 5 * N_SLOTS * CHUNK


def _sc_body(z_hbm, src_hbm, dst_hbm, out_hbm, sidx_v, didx_v,
             srows, drows, out_v, sem_i0, sem_i1, sem_s, sem_d):
    wid = lax.axis_index("s") * NC + lax.axis_index("c")
    lane = lax.iota(jnp.int32, L)
    base_w = wid * PER_W

    ci0 = pltpu.async_copy(src_hbm.at[pl.ds(base_w, PER_W)], sidx_v, sem_i0)
    ci1 = pltpu.async_copy(dst_hbm.at[pl.ds(base_w, PER_W)], didx_v, sem_i1)
    ci0.wait()
    ci1.wait()

    H = CHUNK // 2

    def issue(b, chunk):
        off = chunk * CHUNK
        pltpu.async_copy(
            z_hbm.at[sidx_v.at[pl.ds(off, H)]],
            srows[b].at[pl.ds(0, H)], sem_s[b])
        pltpu.async_copy(
            z_hbm.at[sidx_v.at[pl.ds(off + H, H)]],
            srows[b].at[pl.ds(H, H)], sem_s[b])
        pltpu.async_copy(
            z_hbm.at[didx_v.at[pl.ds(off, H)]],
            drows[b].at[pl.ds(0, H)], sem_d[b])
        pltpu.async_copy(
            z_hbm.at[didx_v.at[pl.ds(off + H, H)]],
            drows[b].at[pl.ds(H, H)], sem_d[b])

    for b in range(N_SLOTS):
        issue(b, b)

    def compute(b, o, chunk):
        cbase = ((o % 5) * N_SLOTS + (chunk - o * N_SLOTS)) * CHUNK

        def group_body(g, carry):
            eids = g * L + lane

            def d_blk(j, accs):
                a0, a1, a2, a3 = accs
                prods = []
                for u in range(16):
                    dv = (lane + (j * 16 + u)) & (D - 1)
                    s = plsc.load_gather(srows[b], [eids, dv])
                    t = plsc.load_gather(drows[b], [eids, dv])
                    prods.append(s * t)
                a0 = a0 + ((prods[0] + prods[1]) + (prods[2] + prods[3]))
                a1 = a1 + ((prods[4] + prods[5]) + (prods[6] + prods[7]))
                a2 = a2 + ((prods[8] + prods[9]) + (prods[10] + prods[11]))
                a3 = a3 + ((prods[12] + prods[13]) + (prods[14] + prods[15]))
                return a0, a1, a2, a3

            z4 = jnp.zeros((L,), jnp.float32)
            a0, a1, a2, a3 = lax.fori_loop(0, D // 16, d_blk,
                                           (z4, z4, z4, z4))
            acc = (a0 + a1) + (a2 + a3)
            out_v[pl.ds(cbase + g * L, L)] = 1.0 / (1.0 + jnp.exp(-acc))
            return carry

        lax.fori_loop(0, CHUNK // L, group_body, 0)

    def outer(o, carry):
        for b in range(N_SLOTS):
            chunk = o * N_SLOTS + b
            pltpu.make_async_copy(
                z_hbm.at[sidx_v.at[pl.ds(0, CHUNK)]], srows[b],
                sem_s[b]).wait()
            pltpu.make_async_copy(
                z_hbm.at[didx_v.at[pl.ds(0, CHUNK)]], drows[b],
                sem_d[b]).wait()
            compute(b, o, chunk)
            nxt = chunk + N_SLOTS

            @pl.when(nxt < N_ITERS)
            def _issue_next():
                issue(b, nxt)

        @pl.when(o % 5 == 4)
        def _flush():
            pltpu.sync_copy(
                out_v, out_hbm.at[pl.ds(base_w + (o // 5) * OUT_W, OUT_W)])

        return carry

    lax.fori_loop(0, N_ITERS // N_SLOTS, outer, 0)


@jax.jit
def _run(z, src, dst):
    mesh = plsc.VectorSubcoreMesh(
        core_axis_name="c", subcore_axis_name="s",
        num_cores=NC, num_subcores=NS)
    kfn = pl.kernel(
        _sc_body,
        out_type=jax.ShapeDtypeStruct((E,), jnp.float32),
        mesh=mesh,
        scratch_types=[
            pltpu.VMEM((PER_W,), jnp.int32),
            pltpu.VMEM((PER_W,), jnp.int32),
            [pltpu.VMEM((CHUNK, D), jnp.float32) for _ in range(N_SLOTS)],
            [pltpu.VMEM((CHUNK, D), jnp.float32) for _ in range(N_SLOTS)],
            pltpu.VMEM((OUT_W,), jnp.float32),
            pltpu.SemaphoreType.DMA,
            pltpu.SemaphoreType.DMA,
            [pltpu.SemaphoreType.DMA for _ in range(N_SLOTS)],
            [pltpu.SemaphoreType.DMA for _ in range(N_SLOTS)],
        ],
        compiler_params=pltpu.CompilerParams(needs_layout_passes=False),
    )
    return kfn(z, src, dst)


def kernel(z, edge_index):
    src = edge_index[0].astype(jnp.int32)
    dst = edge_index[1].astype(jnp.int32)
    return _run(z, src, dst)


# speedup vs baseline: 1.1455x; 1.0005x over previous
import functools

import jax
import jax.numpy as jnp
from jax import lax
from jax.experimental import pallas as pl
from jax.experimental.pallas import tpu as pltpu
from jax.experimental.pallas import tpu_sc as plsc

NC = 2
NS = 16
L = 16
NW = NC * NS

E = 320000
D = 128
PER_W = E // NW
CHUNK = 80
N_ITERS = PER_W // CHUNK
N_SLOTS = 5
OUT_W = 5 * N_SLOTS * CHUNK


def _sc_body(z_hbm, src_hbm, dst_hbm, out_hbm, sidx_v, didx_v,
             srows, drows, out_v, sem_i0, sem_i1, sem_s, sem_d):
    wid = lax.axis_index("s") * NC + lax.axis_index("c")
    lane = lax.iota(jnp.int32, L)
    base_w = wid * PER_W

    ci0 = pltpu.async_copy(src_hbm.at[pl.ds(base_w, PER_W)], sidx_v, sem_i0)
    ci1 = pltpu.async_copy(dst_hbm.at[pl.ds(base_w, PER_W)], didx_v, sem_i1)
    ci0.wait()
    ci1.wait()

    def issue(b, chunk):
        off = chunk * CHUNK
        pltpu.async_copy(
            z_hbm.at[sidx_v.at[pl.ds(off, CHUNK)]], srows[b], sem_s[b])
        pltpu.async_copy(
            z_hbm.at[didx_v.at[pl.ds(off, CHUNK)]], drows[b], sem_d[b])

    for b in range(N_SLOTS):
        issue(b, b)

    def compute(b, o, chunk):
        cbase = ((o % 5) * N_SLOTS + (chunk - o * N_SLOTS)) * CHUNK

        def group_body(g, carry):
            eids = g * L + lane

            def d_blk(j, accs):
                a0, a1, a2, a3 = accs
                prods = []
                for u in range(16):
                    dv = (lane + (j * 16 + u)) & (D - 1)
                    s = plsc.load_gather(srows[b], [eids, dv])
                    t = plsc.load_gather(drows[b], [eids, dv])
                    prods.append(s * t)
                a0 = a0 + ((prods[0] + prods[1]) + (prods[2] + prods[3]))
                a1 = a1 + ((prods[4] + prods[5]) + (prods[6] + prods[7]))
                a2 = a2 + ((prods[8] + prods[9]) + (prods[10] + prods[11]))
                a3 = a3 + ((prods[12] + prods[13]) + (prods[14] + prods[15]))
                return a0, a1, a2, a3

            z4 = jnp.zeros((L,), jnp.float32)
            a0, a1, a2, a3 = lax.fori_loop(0, D // 16, d_blk,
                                           (z4, z4, z4, z4))
            acc = (a0 + a1) + (a2 + a3)
            out_v[pl.ds(cbase + g * L, L)] = 1.0 / (1.0 + jnp.exp(-acc))
            return carry

        lax.fori_loop(0, CHUNK // L, group_body, 0)

    def outer(o, carry):
        for b in range(N_SLOTS):
            chunk = o * N_SLOTS + b
            pltpu.make_async_copy(
                z_hbm.at[sidx_v.at[pl.ds(0, CHUNK)]], srows[b],
                sem_s[b]).wait()
            pltpu.make_async_copy(
                z_hbm.at[didx_v.at[pl.ds(0, CHUNK)]], drows[b],
                sem_d[b]).wait()
            compute(b, o, chunk)
            nxt = chunk + N_SLOTS

            @pl.when(nxt < N_ITERS)
            def _issue_next():
                issue(b, nxt)

        @pl.when(o % 5 == 4)
        def _flush():
            pltpu.sync_copy(
                out_v, out_hbm.at[pl.ds(base_w + (o // 5) * OUT_W, OUT_W)])

        return carry

    lax.fori_loop(0, N_ITERS // N_SLOTS, outer, 0)


@jax.jit
def _run(z, src, dst):
    mesh = plsc.VectorSubcoreMesh(
        core_axis_name="c", subcore_axis_name="s",
        num_cores=NC, num_subcores=NS)
    kfn = pl.kernel(
        _sc_body,
        out_type=jax.ShapeDtypeStruct((E,), jnp.float32),
        mesh=mesh,
        scratch_types=[
            pltpu.VMEM((PER_W,), jnp.int32),
            pltpu.VMEM((PER_W,), jnp.int32),
            [pltpu.VMEM((CHUNK, D), jnp.float32) for _ in range(N_SLOTS)],
            [pltpu.VMEM((CHUNK, D), jnp.float32) for _ in range(N_SLOTS)],
            pltpu.VMEM((OUT_W,), jnp.float32),
            pltpu.SemaphoreType.DMA,
            pltpu.SemaphoreType.DMA,
            [pltpu.SemaphoreType.DMA for _ in range(N_SLOTS)],
            [pltpu.SemaphoreType.DMA for _ in range(N_SLOTS)],
        ],
        compiler_params=pltpu.CompilerParams(needs_layout_passes=False),
    )
    return kfn(z, src, dst)


def kernel(z, edge_index):
    src = edge_index[0].astype(jnp.int32)
    dst = edge_index[1].astype(jnp.int32)
    return _run(z, src, dst)
